# baseline (speedup 1.0000x reference)
import functools

import jax
import jax.numpy as jnp
from jax import lax
from jax.experimental import pallas as pl
from jax.experimental.pallas import tpu as pltpu
from jax.experimental.pallas import tpu_sc as plsc

_INFO = plsc.get_sparse_core_info()
_NC = _INFO.num_cores
_NS = _INFO.num_subcores
_NW = _NC * _NS
_CHUNK = 128


def _make_sc_gather(B, V, D):
    b_per_w = B // _NW
    n_chunks = b_per_w // _CHUNK
    mesh = plsc.VectorSubcoreMesh(core_axis_name="c", subcore_axis_name="s")

    @functools.partial(
        pl.kernel,
        mesh=mesh,
        out_type=jax.ShapeDtypeStruct((B, D), jnp.float32),
        scratch_types=[
            pltpu.VMEM((n_chunks, _CHUNK), jnp.int32),
            pltpu.VMEM((b_per_w, D), jnp.float32),
            pltpu.SemaphoreType.DMA,
        ],
        compiler_params=pltpu.CompilerParams(use_tc_tiling_on_sc=False),
    )
    def k(idx_hbm, table_hbm, out_hbm, idx_v, rows_v, sem):
        wid = lax.axis_index("s") * _NC + lax.axis_index("c")
        pltpu.sync_copy(idx_hbm.at[pl.ds(wid * n_chunks, n_chunks)], idx_v)
        copies = [
            pltpu.async_copy(
                table_hbm.at[idx_v.at[j]],
                rows_v.at[pl.ds(j * _CHUNK, _CHUNK)],
                sem,
            )
            for j in range(n_chunks)
        ]
        for c in copies:
            c.wait()
        pltpu.sync_copy(rows_v, out_hbm.at[pl.ds(wid * b_per_w, b_per_w)])

    return k


def kernel(agent_ids, table):
    B = agent_ids.shape[0]
    V, D = table.shape
    idx2d = agent_ids.astype(jnp.int32).reshape(B // _CHUNK, _CHUNK)
    return _make_sc_gather(B, V, D)(idx2d, table)


# speedup vs baseline: 1.8180x; 1.8180x over previous
import functools

import jax
import jax.numpy as jnp
from jax import lax
from jax.experimental import pallas as pl
from jax.experimental.pallas import tpu as pltpu
from jax.experimental.pallas import tpu_sc as plsc

_INFO = plsc.get_sparse_core_info()
_NC = _INFO.num_cores
_NS = _INFO.num_subcores
_NW = _NC * _NS

_TA = 128
_TF = 8
_NBUF = 4


def _make_clone(V, D):
    a_tiles = -(-V // _TA)
    a_full = V // _TA
    v_tail = V - a_full * _TA
    g_tiles = D // _TF
    n_tiles = g_tiles * a_tiles
    mesh = plsc.VectorSubcoreMesh(core_axis_name="c", subcore_axis_name="s")
    max_steps = -(-n_tiles // _NW)
    outer = -(-max_steps // _NBUF)

    @functools.partial(
        pl.kernel,
        mesh=mesh,
        out_type=jax.ShapeDtypeStruct((n_tiles, _TF, _TA), jnp.float32),
        scratch_types=[
            [pltpu.VMEM((_TF, _TA), jnp.float32) for _ in range(_NBUF)],
            [pltpu.SemaphoreType.DMA for _ in range(_NBUF)],
            [pltpu.SemaphoreType.DMA for _ in range(_NBUF)],
        ],
    )
    def clone(tab_hbm, tail_hbm, out_hbm, bufs, in_sems, out_sems):
        w = lax.axis_index("s") * _NC + lax.axis_index("c")

        def tile_of(r, b):
            return w + (r * _NBUF + b) * _NW

        def start_in(t, b):
            g = t // a_tiles
            a = t % a_tiles

            @pl.when(a < a_full)
            def _():
                pltpu.async_copy(
                    tab_hbm.at[pl.ds(g * _TF, _TF), pl.ds(a * _TA, _TA)],
                    bufs[b],
                    in_sems[b],
                )

            @pl.when(a >= a_full)
            def _():
                pltpu.async_copy(
                    tail_hbm.at[pl.ds(g * _TF, _TF), :],
                    bufs[b],
                    in_sems[b],
                )

        def wait_in(t, b):
            g = t // a_tiles
            a = t % a_tiles

            @pl.when(a < a_full)
            def _():
                pltpu.make_async_copy(
                    tab_hbm.at[pl.ds(g * _TF, _TF), pl.ds(a * _TA, _TA)],
                    bufs[b],
                    in_sems[b],
                ).wait()

            @pl.when(a >= a_full)
            def _():
                pltpu.make_async_copy(
                    tail_hbm.at[pl.ds(g * _TF, _TF), :],
                    bufs[b],
                    in_sems[b],
                ).wait()

        def start_out(t, b):
            pltpu.async_copy(bufs[b], out_hbm.at[t], out_sems[b])

        def wait_out(t, b):
            pltpu.make_async_copy(bufs[b], out_hbm.at[t], out_sems[b]).wait()

        def body(r, _):
            for b in range(_NBUF):
                t = tile_of(r, b)

                @pl.when(jnp.logical_and(r > 0, t < n_tiles))
                def _():
                    wait_out(tile_of(r - 1, b), b)

                @pl.when(t < n_tiles)
                def _():
                    start_in(t, b)

            for b in range(_NBUF):
                t = tile_of(r, b)

                @pl.when(t < n_tiles)
                def _():
                    wait_in(t, b)
                    start_out(t, b)

            return 0

        lax.fori_loop(0, outer, body, 0, unroll=False)
        for b in range(_NBUF):
            t = tile_of(outer - 1, b)

            @pl.when(t < n_tiles)
            def _():
                wait_out(t, b)

    return clone


def _make_gather(B, V, D):
    a_tiles = -(-V // _TA)
    n_tiles = (D // _TF) * a_tiles
    flat_n = n_tiles * _TF * _TA
    mesh = plsc.VectorSubcoreMesh(core_axis_name="c", subcore_axis_name="s")
    assert D == _NW

    @functools.partial(
        pl.kernel,
        mesh=mesh,
        out_type=jax.ShapeDtypeStruct((D, B), jnp.float32),
        scratch_types=[
            pltpu.VMEM((B,), jnp.int32),
            pltpu.VMEM((B,), jnp.int32),
            pltpu.VMEM((B,), jnp.float32),
            pltpu.SemaphoreType.DMA,
        ],
        compiler_params=pltpu.CompilerParams(use_tc_tiling_on_sc=False),
    )
    def gather(idx_hbm, flat_hbm, out_hbm, idx_v, off_v, vals_v, sem):
        w = lax.axis_index("s") * _NC + lax.axis_index("c")
        pltpu.sync_copy(idx_hbm, idx_v)
        base = ((w // _TF) * a_tiles) * 1024 + (w % _TF) * _TA

        def body(j, _):
            i16 = idx_v[pl.ds(j * 16, 16)]
            off = base + ((i16 >> 7) << 10) + (i16 & 127)
            off_v[pl.ds(j * 16, 16)] = off
            return 0

        lax.fori_loop(0, B // 16, body, 0, unroll=8)
        pltpu.async_copy(flat_hbm.at[off_v], vals_v, sem).wait()
        pltpu.sync_copy(vals_v, out_hbm.at[w])

    return gather


def kernel(agent_ids, table):
    B = agent_ids.shape[0]
    V, D = table.shape
    idx = agent_ids.astype(jnp.int32)
    a_full = V // _TA
    tail_pad = jnp.zeros((D, _TA), jnp.float32)
    tail_pad = tail_pad.at[:, : V - a_full * _TA].set(table[a_full * _TA :].T)
    clone3 = _make_clone(V, D)(table.T, tail_pad)
    flat = clone3.reshape(-1)
    out_t = _make_gather(B, V, D)(idx, flat)
    return out_t.T


# speedup vs baseline: 3.8837x; 2.1363x over previous
import functools

import jax
import jax.numpy as jnp
from jax import lax
from jax.experimental import pallas as pl
from jax.experimental.pallas import tpu as pltpu
from jax.experimental.pallas import tpu_sc as plsc

_INFO = plsc.get_sparse_core_info()
_NC = _INFO.num_cores
_NS = _INFO.num_subcores
_NW = _NC * _NS

_TA = 128
_TF = 8
_KC = 601


def _tc_clone_kernel(in_ref, out_ref):
    for k in range(_KC):
        out_ref[pl.ds(k * _TF, _TF), :] = in_ref[:, pl.ds(k * _TA, _TA)]


def _make_clone(V, D):
    a_tiles = -(-V // _TA)
    g_tiles = D // _TF
    n_tiles = g_tiles * a_tiles
    chunks = a_tiles // _KC

    return pl.pallas_call(
        _tc_clone_kernel,
        grid=(g_tiles, chunks),
        in_specs=[
            pl.BlockSpec((_TF, _KC * _TA), lambda g, c: (g, c)),
        ],
        out_specs=pl.BlockSpec((_KC * _TF, _TA), lambda g, c: (g * chunks + c, 0)),
        out_shape=jax.ShapeDtypeStruct((n_tiles * _TF, _TA), jnp.float32),
    )


def _make_gather(B, V, D):
    a_tiles = -(-V // _TA)
    mesh = plsc.VectorSubcoreMesh(core_axis_name="c", subcore_axis_name="s")
    assert D == _NW

    @functools.partial(
        pl.kernel,
        mesh=mesh,
        out_type=jax.ShapeDtypeStruct((D, B), jnp.float32),
        scratch_types=[
            pltpu.VMEM((B,), jnp.int32),
            pltpu.VMEM((B,), jnp.int32),
            pltpu.VMEM((B,), jnp.float32),
            pltpu.SemaphoreType.DMA,
        ],
        compiler_params=pltpu.CompilerParams(use_tc_tiling_on_sc=False),
    )
    def gather(idx_hbm, flat_hbm, out_hbm, idx_v, off_v, vals_v, sem):
        w = lax.axis_index("s") * _NC + lax.axis_index("c")
        pltpu.sync_copy(idx_hbm, idx_v)
        base = ((w // _TF) * a_tiles) * 1024 + (w % _TF) * _TA

        def body(j, _):
            i16 = idx_v[pl.ds(j * 16, 16)]
            off = base + ((i16 >> 7) << 10) + (i16 & 127)
            off_v[pl.ds(j * 16, 16)] = off
            return 0

        lax.fori_loop(0, B // 16, body, 0, unroll=8)
        pltpu.async_copy(flat_hbm.at[off_v], vals_v, sem).wait()
        pltpu.sync_copy(vals_v, out_hbm.at[w])

    return gather


def kernel(agent_ids, table):
    B = agent_ids.shape[0]
    V, D = table.shape
    idx = agent_ids.astype(jnp.int32)
    clone2 = _make_clone(V, D)(table.T)
    flat = clone2.reshape(-1)
    out_t = _make_gather(B, V, D)(idx, flat)
    return out_t.T


# speedup vs baseline: 4.0374x; 1.0396x over previous
import functools

import jax
import jax.numpy as jnp
from jax import lax
from jax.experimental import pallas as pl
from jax.experimental.pallas import tpu as pltpu
from jax.experimental.pallas import tpu_sc as plsc

_INFO = plsc.get_sparse_core_info()
_NC = _INFO.num_cores
_NS = _INFO.num_subcores
_NW = _NC * _NS

_TA = 128
_TF = 8
_KC = 601


def _tc_clone_kernel(in_ref, out_ref):
    for k in range(_KC):
        out_ref[pl.ds(k * _TF, _TF), :] = in_ref[:, pl.ds(k * _TA, _TA)]


def _make_clone_g(V, D, g):
    a_tiles = -(-V // _TA)
    chunks = a_tiles // _KC

    return pl.pallas_call(
        _tc_clone_kernel,
        grid=(chunks,),
        in_specs=[
            pl.BlockSpec((_TF, _KC * _TA), lambda c: (g, c)),
        ],
        out_specs=pl.BlockSpec((_KC * _TF, _TA), lambda c: (c, 0)),
        out_shape=jax.ShapeDtypeStruct((a_tiles * _TF, _TA), jnp.float32),
    )


def _make_gather_g(B, V, D):
    a_tiles = -(-V // _TA)
    mesh = plsc.VectorSubcoreMesh(core_axis_name="c", subcore_axis_name="s")
    quarters = _NW // _TF
    bq = B // quarters

    @functools.partial(
        pl.kernel,
        mesh=mesh,
        out_type=jax.ShapeDtypeStruct((_TF, B), jnp.float32),
        scratch_types=[
            pltpu.VMEM((bq,), jnp.int32),
            pltpu.VMEM((bq,), jnp.int32),
            pltpu.VMEM((bq,), jnp.float32),
            pltpu.SemaphoreType.DMA,
        ],
        compiler_params=pltpu.CompilerParams(use_tc_tiling_on_sc=False),
    )
    def gather(idx_hbm, flat_hbm, out_hbm, idx_v, off_v, vals_v, sem):
        w = lax.axis_index("s") * _NC + lax.axis_index("c")
        f8 = w // quarters
        q = w % quarters
        pltpu.sync_copy(idx_hbm.at[pl.ds(q * bq, bq)], idx_v)
        base = f8 * _TA

        def body(j, _):
            i16 = idx_v[pl.ds(j * 16, 16)]
            off = base + ((i16 >> 7) << 10) + (i16 & 127)
            off_v[pl.ds(j * 16, 16)] = off
            return 0

        lax.fori_loop(0, bq // 16, body, 0, unroll=8)
        pltpu.async_copy(flat_hbm.at[off_v], vals_v, sem).wait()
        pltpu.sync_copy(vals_v, out_hbm.at[f8, pl.ds(q * bq, bq)])

    return gather


def kernel(agent_ids, table):
    B = agent_ids.shape[0]
    V, D = table.shape
    idx = agent_ids.astype(jnp.int32)
    tab_t = table.T
    gather_g = _make_gather_g(B, V, D)
    outs = []
    for g in range(D // _TF):
        clone_g = _make_clone_g(V, D, g)(tab_t)
        outs.append(gather_g(idx, clone_g.reshape(-1)))
    out_t = jnp.concatenate(outs, axis=0)
    return out_t.T


# speedup vs baseline: 4.0635x; 1.0065x over previous
import functools

import jax
import jax.numpy as jnp
from jax import lax
from jax.experimental import pallas as pl
from jax.experimental.pallas import tpu as pltpu
from jax.experimental.pallas import tpu_sc as plsc

_INFO = plsc.get_sparse_core_info()
_NC = _INFO.num_cores
_NS = _INFO.num_subcores
_NW = _NC * _NS

_TA = 128
_TF = 8
_KC = 601


def _tc_clone_kernel(in_ref, out_ref):
    for k in range(_KC):
        out_ref[pl.ds(k * _TF, _TF), :] = in_ref[:, pl.ds(k * _TA, _TA)]


def _make_clone_g(V, D, g):
    a_tiles = -(-V // _TA)
    chunks = a_tiles // _KC

    return pl.pallas_call(
        _tc_clone_kernel,
        grid=(chunks,),
        in_specs=[
            pl.BlockSpec((_TF, _KC * _TA), lambda c: (g, c)),
        ],
        out_specs=pl.BlockSpec((_KC * _TF, _TA), lambda c: (c, 0)),
        out_shape=jax.ShapeDtypeStruct((a_tiles * _TF, _TA), jnp.float32),
    )


def _make_gather_g(B, V, D):
    a_tiles = -(-V // _TA)
    mesh = plsc.VectorSubcoreMesh(core_axis_name="c", subcore_axis_name="s")
    quarters = _NW // _TF
    bq = B // quarters

    @functools.partial(
        pl.kernel,
        mesh=mesh,
        out_type=jax.ShapeDtypeStruct((_TF, B), jnp.float32),
        scratch_types=[
            pltpu.VMEM((bq,), jnp.int32),
            pltpu.VMEM((bq,), jnp.int32),
            pltpu.VMEM((bq,), jnp.float32),
            pltpu.SemaphoreType.DMA,
        ],
        compiler_params=pltpu.CompilerParams(use_tc_tiling_on_sc=False),
    )
    def gather(idx_hbm, flat_hbm, out_hbm, idx_v, off_v, vals_v, sem):
        w = lax.axis_index("s") * _NC + lax.axis_index("c")
        f8 = w // quarters
        q = w % quarters
        pltpu.sync_copy(idx_hbm.at[pl.ds(q * bq, bq)], idx_v)
        base = f8 * _TA

        def body(j, _):
            i16 = idx_v[pl.ds(j * 16, 16)]
            off = base + ((i16 >> 7) << 10) + (i16 & 127)
            off_v[pl.ds(j * 16, 16)] = off
            return 0

        half = bq // 2
        lax.fori_loop(0, half // 16, body, 0, unroll=8)
        c1 = pltpu.async_copy(
            flat_hbm.at[off_v.at[pl.ds(0, half)]],
            vals_v.at[pl.ds(0, half)],
            sem,
        )
        lax.fori_loop(half // 16, bq // 16, body, 0, unroll=8)
        c2 = pltpu.async_copy(
            flat_hbm.at[off_v.at[pl.ds(half, half)]],
            vals_v.at[pl.ds(half, half)],
            sem,
        )
        c1.wait()
        c2.wait()
        pltpu.sync_copy(vals_v, out_hbm.at[f8, pl.ds(q * bq, bq)])

    return gather


def kernel(agent_ids, table):
    B = agent_ids.shape[0]
    V, D = table.shape
    idx = agent_ids.astype(jnp.int32)
    tab_t = table.T
    gather_g = _make_gather_g(B, V, D)
    outs = []
    for g in range(D // _TF):
        clone_g = _make_clone_g(V, D, g)(tab_t)
        outs.append(gather_g(idx, clone_g.reshape(-1)))
    out_t = jnp.concatenate(outs, axis=0)
    return out_t.T
